# baseline (speedup 1.0000x reference)
import functools

import jax
import jax.numpy as jnp
from jax.experimental import pallas as pl
from jax.experimental.pallas import tpu as pltpu
from jax.experimental.pallas import tpu_sc as plsc

LEVELS = 4
NUM_EMB = 8192
EMB_DIM = 256
BETA = 0.25
B = 4096

TB = 256
KB = 8192
T_BLOCKS = B // TB
K_TILES = NUM_EMB // KB


def _argmin_kernel(r_ref, emb_ref, zsq_ref, idx_ref, esq_ref):
    t = pl.program_id(0)

    @pl.when(t == 0)
    def _compute_esq():
        for k in range(K_TILES):
            emb_tile = emb_ref[0, pl.ds(k * KB, KB), :]
            s = jnp.sum(emb_tile * emb_tile, axis=1, keepdims=True)
            esq_ref[0:1, pl.ds(k * KB, KB)] = jnp.transpose(s)

    r = r_ref[...]
    z_sq = jnp.transpose(zsq_ref[0, :, :])

    best_min = jnp.full((TB, 1), jnp.inf, dtype=jnp.float32)
    best_idx = jnp.zeros((TB, 1), dtype=jnp.int32)
    for k in range(K_TILES):
        emb_tile = emb_ref[0, pl.ds(k * KB, KB), :]
        m = jax.lax.dot_general(r, emb_tile, (((1,), (1,)), ((), ())),
                                preferred_element_type=jnp.float32)
        e_sq = esq_ref[0:1, pl.ds(k * KB, KB)]
        dist = (m * -2.0 + z_sq) + e_sq
        tile_arg = (jnp.argmin(dist, axis=1).astype(jnp.int32)[:, None]
                    + k * KB)
        if K_TILES == 1:
            best_idx = tile_arg
        else:
            tile_min = jnp.min(dist, axis=1, keepdims=True)
            better = tile_min < best_min
            best_min = jnp.where(better, tile_min, best_min)
            best_idx = jnp.where(better, tile_arg, best_idx)

    idx_ref[0, 0, :] = jnp.reshape(best_idx, (TB,))


def _tc_argmin(r, codebooks, z_sq, lvl):
    idx3 = pl.pallas_call(
        _argmin_kernel,
        grid=(T_BLOCKS,),
        in_specs=[
            pl.BlockSpec((TB, EMB_DIM), lambda t: (t, 0)),
            pl.BlockSpec((1, NUM_EMB, EMB_DIM), lambda t, _l=lvl: (_l, 0, 0)),
            pl.BlockSpec((1, 1, TB), lambda t: (t, 0, 0)),
        ],
        out_specs=pl.BlockSpec((1, 1, TB), lambda t: (t, 0, 0)),
        out_shape=jax.ShapeDtypeStruct((T_BLOCKS, 1, TB), jnp.int32),
        scratch_shapes=[pltpu.VMEM((1, NUM_EMB), jnp.float32)],
    )(r, codebooks, z_sq.reshape(T_BLOCKS, 1, TB))
    return jnp.reshape(idx3, (B,))


_SC_INFO = plsc.get_sparse_core_info()
_NC = _SC_INFO.num_cores
_NW = _SC_INFO.num_cores * _SC_INFO.num_subcores
_BPW = B // _NW


def _sc_gather(table, idx):
    mesh = plsc.VectorSubcoreMesh(core_axis_name="c", subcore_axis_name="s")

    @functools.partial(
        pl.kernel, mesh=mesh,
        out_type=jax.ShapeDtypeStruct((B, EMB_DIM), jnp.float32),
        scratch_types=[
            pltpu.VMEM((_BPW,), jnp.int32),
            pltpu.VMEM((_BPW, EMB_DIM), jnp.float32),
            pltpu.SemaphoreType.DMA,
        ],
    )
    def gk(table_hbm, idx_hbm, out_hbm, idx_v, rows_v, sem):
        wid = jax.lax.axis_index("s") * _NC + jax.lax.axis_index("c")
        base = wid * _BPW
        pltpu.sync_copy(idx_hbm.at[pl.ds(base, _BPW)], idx_v)
        pltpu.async_copy(table_hbm.at[idx_v], rows_v, sem).wait()
        pltpu.sync_copy(rows_v, out_hbm.at[pl.ds(base, _BPW)])

    return gk(table, idx)


@jax.jit
def kernel(z, codebooks):
    table = jnp.reshape(codebooks, (LEVELS * NUM_EMB, EMB_DIM))
    residual = z
    quant_sum = jnp.zeros_like(z)
    all_indices = []
    total_vq_loss = jnp.asarray(0.0, dtype=jnp.float32)
    for lvl in range(LEVELS):
        z_sq = jnp.sum(residual * residual, axis=1, keepdims=True)
        idx = _tc_argmin(residual, codebooks, z_sq, lvl)
        z_q = _sc_gather(table, idx + (lvl * NUM_EMB))
        t = z_q - residual
        m = jnp.mean(t * t)
        c = residual + t
        quant_sum = quant_sum + c
        residual = residual - c
        residual, m = jax.lax.optimization_barrier((residual, m))
        total_vq_loss = total_vq_loss + (m + BETA * m)
        all_indices.append(idx)
    return quant_sum, jnp.stack(all_indices, axis=0), total_vq_loss


# speedup vs baseline: 1.1142x; 1.1142x over previous
import functools

import jax
import jax.numpy as jnp
from jax.experimental import pallas as pl
from jax.experimental.pallas import tpu as pltpu
from jax.experimental.pallas import tpu_sc as plsc

LEVELS = 4
NUM_EMB = 8192
EMB_DIM = 256
BETA = 0.25
B = 4096

TB = 512
KB = 8192
T_BLOCKS = B // TB
K_TILES = NUM_EMB // KB


def _argmin_kernel(r_ref, emb_ref, zsq_ref, idx_ref, esq_ref):
    t = pl.program_id(0)

    @pl.when(t == 0)
    def _compute_esq():
        for k in range(K_TILES):
            emb_tile = emb_ref[0, pl.ds(k * KB, KB), :]
            s = jnp.sum(emb_tile * emb_tile, axis=1, keepdims=True)
            esq_ref[0:1, pl.ds(k * KB, KB)] = jnp.transpose(s)

    r = r_ref[...]
    z_sq = jnp.transpose(zsq_ref[0, :, :])

    best_min = jnp.full((TB, 1), jnp.inf, dtype=jnp.float32)
    best_idx = jnp.zeros((TB, 1), dtype=jnp.int32)
    for k in range(K_TILES):
        emb_tile = emb_ref[0, pl.ds(k * KB, KB), :]
        m = jax.lax.dot_general(r, emb_tile, (((1,), (1,)), ((), ())),
                                preferred_element_type=jnp.float32)
        e_sq = esq_ref[0:1, pl.ds(k * KB, KB)]
        dist = (m * -2.0 + z_sq) + e_sq
        tile_arg = (jnp.argmin(dist, axis=1).astype(jnp.int32)[:, None]
                    + k * KB)
        if K_TILES == 1:
            best_idx = tile_arg
        else:
            tile_min = jnp.min(dist, axis=1, keepdims=True)
            better = tile_min < best_min
            best_min = jnp.where(better, tile_min, best_min)
            best_idx = jnp.where(better, tile_arg, best_idx)

    idx_ref[0, 0, :] = jnp.reshape(best_idx, (TB,))


def _tc_argmin(r, codebooks, z_sq, lvl):
    idx3 = pl.pallas_call(
        _argmin_kernel,
        grid=(T_BLOCKS,),
        in_specs=[
            pl.BlockSpec((TB, EMB_DIM), lambda t: (t, 0)),
            pl.BlockSpec((1, NUM_EMB, EMB_DIM), lambda t, _l=lvl: (_l, 0, 0)),
            pl.BlockSpec((1, 1, TB), lambda t: (t, 0, 0)),
        ],
        out_specs=pl.BlockSpec((1, 1, TB), lambda t: (t, 0, 0)),
        out_shape=jax.ShapeDtypeStruct((T_BLOCKS, 1, TB), jnp.int32),
        scratch_shapes=[pltpu.VMEM((1, NUM_EMB), jnp.float32)],
    )(r, codebooks, z_sq.reshape(T_BLOCKS, 1, TB))
    return jnp.reshape(idx3, (B,))


_SC_INFO = plsc.get_sparse_core_info()
_NC = _SC_INFO.num_cores
_NW = _SC_INFO.num_cores * _SC_INFO.num_subcores
_BPW = B // _NW


def _sc_gather(table, idx):
    mesh = plsc.VectorSubcoreMesh(core_axis_name="c", subcore_axis_name="s")

    @functools.partial(
        pl.kernel, mesh=mesh,
        out_type=jax.ShapeDtypeStruct((B, EMB_DIM), jnp.float32),
        scratch_types=[
            pltpu.VMEM((_BPW,), jnp.int32),
            pltpu.VMEM((_BPW, EMB_DIM), jnp.float32),
            pltpu.SemaphoreType.DMA,
        ],
    )
    def gk(table_hbm, idx_hbm, out_hbm, idx_v, rows_v, sem):
        wid = jax.lax.axis_index("s") * _NC + jax.lax.axis_index("c")
        base = wid * _BPW
        pltpu.sync_copy(idx_hbm.at[pl.ds(base, _BPW)], idx_v)
        pltpu.async_copy(table_hbm.at[idx_v], rows_v, sem).wait()
        pltpu.sync_copy(rows_v, out_hbm.at[pl.ds(base, _BPW)])

    return gk(table, idx)


@jax.jit
def kernel(z, codebooks):
    table = jnp.reshape(codebooks, (LEVELS * NUM_EMB, EMB_DIM))
    residual = z
    quant_sum = jnp.zeros_like(z)
    all_indices = []
    total_vq_loss = jnp.asarray(0.0, dtype=jnp.float32)
    for lvl in range(LEVELS):
        z_sq = jnp.sum(residual * residual, axis=1, keepdims=True)
        idx = _tc_argmin(residual, codebooks, z_sq, lvl)
        z_q = _sc_gather(table, idx + (lvl * NUM_EMB))
        t = z_q - residual
        m = jnp.mean(t * t)
        c = residual + t
        quant_sum = quant_sum + c
        residual = residual - c
        total_vq_loss = total_vq_loss + (m + BETA * m)
        all_indices.append(idx)
    return quant_sum, jnp.stack(all_indices, axis=0), total_vq_loss
